# baseline (speedup 1.0000x reference)
import functools

import jax
import jax.numpy as jnp
from jax import lax
from jax.experimental import pallas as pl
from jax.experimental.pallas import tpu as pltpu
from jax.experimental.pallas import tpu_sc as plsc

NC = 2
NS = 16
NW = NC * NS
CHUNK = 128
ROW = 128


TR_LANES = 16384


def _tr_body(tt_ref, out_ref):
    x = tt_ref[...]
    for q in range(TR_LANES // 512):
        y = jnp.concatenate(
            [x[:, 512 * q + 128 * p:512 * q + 128 * (p + 1)]
             for p in range(4)], axis=0)
        out_ref[128 * q:128 * (q + 1), :] = jnp.transpose(y)


def _transpose_table(table_t):
    n_feat, vocab = table_t.shape
    grid = (pl.cdiv(vocab, TR_LANES),)
    out_rows = grid[0] * (TR_LANES // 4)
    return pl.pallas_call(
        _tr_body,
        grid=grid,
        in_specs=[pl.BlockSpec((n_feat, TR_LANES), lambda u: (0, u))],
        out_specs=pl.BlockSpec((TR_LANES // 4, ROW), lambda u: (u, 0)),
        out_shape=jax.ShapeDtypeStruct((out_rows, ROW), jnp.float32),
    )(table_t)


def _make_sc_gather(batch):
    b_per_w = batch // NW
    n_chunks = b_per_w // CHUNK
    mesh = plsc.VectorSubcoreMesh(core_axis_name="c", subcore_axis_name="s")

    @functools.partial(
        pl.kernel,
        mesh=mesh,
        out_type=jax.ShapeDtypeStruct((NW, b_per_w, ROW), jnp.float32),
        scratch_types=[
            pltpu.VMEM((n_chunks, CHUNK), jnp.int32),
            pltpu.VMEM((b_per_w, ROW), jnp.float32),
            pltpu.SemaphoreType.DMA,
        ],
    )
    def gather(table_hbm, idx_hbm, out_hbm, idx_v, rows_v, sem):
        wid = lax.axis_index("s") * NC + lax.axis_index("c")
        pltpu.sync_copy(idx_hbm.at[wid], idx_v)
        copies = [
            pltpu.async_copy(
                table_hbm.at[idx_v.at[j]],
                rows_v.at[pl.ds(j * CHUNK, CHUNK)],
                sem,
            )
            for j in range(n_chunks)
        ]
        for c in copies:
            c.wait()
        pltpu.sync_copy(rows_v, out_hbm.at[wid])

    return gather


def _mlp_body(xin_ref, x4_ref, w1x_ref, w1e_ref, b1_ref, w2_ref, b2_ref,
              out_ref):
    c0 = xin_ref[..., 0:1]
    s1 = jnp.floor(c0 * (1.0 / 128.0))
    sub = s1 - 4.0 * jnp.floor(s1 * 0.25)
    x4 = x4_ref[...]
    emb_dim = w1e_ref.shape[0]
    xemb = jnp.zeros((x4.shape[0], emb_dim), jnp.float32)
    for s in range(4):
        sel = sub == float(s)
        xemb = xemb + jnp.where(sel, x4[:, s * emb_dim:(s + 1) * emb_dim], 0.0)
    h = jnp.dot(xin_ref[...], w1x_ref[...], preferred_element_type=jnp.float32)
    h = h + jnp.dot(xemb, w1e_ref[...], preferred_element_type=jnp.float32)
    h = jnp.maximum(h + b1_ref[...], 0.0)
    o = jnp.dot(h, w2_ref[...], preferred_element_type=jnp.float32)
    out_ref[...] = jnp.maximum(o + b2_ref[...], 0.0)


def _mlp(inputs, x4, w1x, w1e, b1, w2, b2, block_m=1024):
    batch, n_feat = inputs.shape
    hidden = w1x.shape[1]
    out_dim = w2.shape[1]
    grid = (batch // block_m,)
    return pl.pallas_call(
        _mlp_body,
        grid=grid,
        in_specs=[
            pl.BlockSpec((block_m, n_feat), lambda i: (i, 0)),
            pl.BlockSpec((block_m, ROW), lambda i: (i, 0)),
            pl.BlockSpec((n_feat, hidden), lambda i: (0, 0)),
            pl.BlockSpec((w1e.shape[0], hidden), lambda i: (0, 0)),
            pl.BlockSpec((1, hidden), lambda i: (0, 0)),
            pl.BlockSpec((hidden, out_dim), lambda i: (0, 0)),
            pl.BlockSpec((1, out_dim), lambda i: (0, 0)),
        ],
        out_specs=pl.BlockSpec((block_m, out_dim), lambda i: (i, 0)),
        out_shape=jax.ShapeDtypeStruct((batch, out_dim), jnp.float32),
    )(inputs, x4, w1x, w1e, b1, w2, b2)


def kernel(inputs, emb_table, W1, b1, W2, b2):
    batch, n_feat = inputs.shape
    vocab, emb_dim = emb_table.shape
    hidden = W1.shape[1]

    idx = inputs[:, 0].astype(jnp.int32)
    hi = (((idx >> 9) << 7) | (idx & 127)).reshape(
        NW, batch // NW // CHUNK, CHUNK)
    table4 = _transpose_table(emb_table.T)
    x4 = _make_sc_gather(batch)(table4, hi)
    x4 = x4.reshape(batch, ROW)

    w1x = jnp.concatenate(
        [jnp.zeros((1, hidden), jnp.float32), W1[: n_feat - 1]], axis=0)
    w1e = W1[n_feat - 1:]
    return _mlp(inputs, x4, w1x, w1e, b1[None, :], W2, b2[None, :])


# speedup vs baseline: 1.1059x; 1.1059x over previous
import functools

import jax
import jax.numpy as jnp
from jax import lax
from jax.experimental import pallas as pl
from jax.experimental.pallas import tpu as pltpu
from jax.experimental.pallas import tpu_sc as plsc

NC = 2
NS = 16
NW = NC * NS
CHUNK = 128
ROW = 128


TR_LANES = 32768


def _tr_body(tt_ref, out_ref):
    x = tt_ref[...]
    for q in range(TR_LANES // 512):
        y = jnp.concatenate(
            [x[:, 512 * q + 128 * p:512 * q + 128 * (p + 1)]
             for p in range(4)], axis=0)
        out_ref[128 * q:128 * (q + 1), :] = jnp.transpose(y)


def _transpose_table(table_t):
    n_feat, vocab = table_t.shape
    grid = (pl.cdiv(vocab, TR_LANES),)
    out_rows = grid[0] * (TR_LANES // 4)
    return pl.pallas_call(
        _tr_body,
        grid=grid,
        in_specs=[pl.BlockSpec((n_feat, TR_LANES), lambda u: (0, u))],
        out_specs=pl.BlockSpec((TR_LANES // 4, ROW), lambda u: (u, 0)),
        out_shape=jax.ShapeDtypeStruct((out_rows, ROW), jnp.float32),
    )(table_t)


def _make_sc_gather(batch):
    b_per_w = batch // NW
    n_chunks = b_per_w // CHUNK
    mesh = plsc.VectorSubcoreMesh(core_axis_name="c", subcore_axis_name="s")

    @functools.partial(
        pl.kernel,
        mesh=mesh,
        out_type=jax.ShapeDtypeStruct((NW, b_per_w, ROW), jnp.float32),
        scratch_types=[
            pltpu.VMEM((n_chunks, CHUNK), jnp.int32),
            pltpu.VMEM((b_per_w, ROW), jnp.float32),
            pltpu.SemaphoreType.DMA,
        ],
    )
    def gather(table_hbm, idx_hbm, out_hbm, idx_v, rows_v, sem):
        wid = lax.axis_index("s") * NC + lax.axis_index("c")
        pltpu.sync_copy(idx_hbm.at[wid], idx_v)
        copies = [
            pltpu.async_copy(
                table_hbm.at[idx_v.at[j]],
                rows_v.at[pl.ds(j * CHUNK, CHUNK)],
                sem,
            )
            for j in range(n_chunks)
        ]
        for c in copies:
            c.wait()
        pltpu.sync_copy(rows_v, out_hbm.at[wid])

    return gather


def _mlp_body(xin_ref, x4_ref, w1x_ref, w1e_ref, b1_ref, w2_ref, b2_ref,
              out_ref):
    c0 = xin_ref[..., 0:1]
    s1 = jnp.floor(c0 * (1.0 / 128.0))
    sub = s1 - 4.0 * jnp.floor(s1 * 0.25)
    x4 = x4_ref[...]
    emb_dim = w1e_ref.shape[0]
    xemb = jnp.zeros((x4.shape[0], emb_dim), jnp.float32)
    for s in range(4):
        sel = sub == float(s)
        xemb = xemb + jnp.where(sel, x4[:, s * emb_dim:(s + 1) * emb_dim], 0.0)
    h = jnp.dot(xin_ref[...], w1x_ref[...], preferred_element_type=jnp.float32)
    h = h + jnp.dot(xemb, w1e_ref[...], preferred_element_type=jnp.float32)
    h = jnp.maximum(h + b1_ref[...], 0.0)
    o = jnp.dot(h, w2_ref[...], preferred_element_type=jnp.float32)
    out_ref[...] = jnp.maximum(o + b2_ref[...], 0.0)


def _mlp(inputs, x4, w1x, w1e, b1, w2, b2, block_m=1024):
    batch, n_feat = inputs.shape
    hidden = w1x.shape[1]
    out_dim = w2.shape[1]
    grid = (batch // block_m,)
    return pl.pallas_call(
        _mlp_body,
        grid=grid,
        in_specs=[
            pl.BlockSpec((block_m, n_feat), lambda i: (i, 0)),
            pl.BlockSpec((block_m, ROW), lambda i: (i, 0)),
            pl.BlockSpec((n_feat, hidden), lambda i: (0, 0)),
            pl.BlockSpec((w1e.shape[0], hidden), lambda i: (0, 0)),
            pl.BlockSpec((1, hidden), lambda i: (0, 0)),
            pl.BlockSpec((hidden, out_dim), lambda i: (0, 0)),
            pl.BlockSpec((1, out_dim), lambda i: (0, 0)),
        ],
        out_specs=pl.BlockSpec((block_m, out_dim), lambda i: (i, 0)),
        out_shape=jax.ShapeDtypeStruct((batch, out_dim), jnp.float32),
    )(inputs, x4, w1x, w1e, b1, w2, b2)


def kernel(inputs, emb_table, W1, b1, W2, b2):
    batch, n_feat = inputs.shape
    vocab, emb_dim = emb_table.shape
    hidden = W1.shape[1]

    idx = inputs[:, 0].astype(jnp.int32)
    hi = (((idx >> 9) << 7) | (idx & 127)).reshape(
        NW, batch // NW // CHUNK, CHUNK)
    table4 = _transpose_table(emb_table.T)
    x4 = _make_sc_gather(batch)(table4, hi)
    x4 = x4.reshape(batch, ROW)

    w1x = jnp.concatenate(
        [jnp.zeros((1, hidden), jnp.float32), W1[: n_feat - 1]], axis=0)
    w1e = W1[n_feat - 1:]
    return _mlp(inputs, x4, w1x, w1e, b1[None, :], W2, b2[None, :])


# speedup vs baseline: 1.1167x; 1.0097x over previous
import functools

import jax
import jax.numpy as jnp
from jax import lax
from jax.experimental import pallas as pl
from jax.experimental.pallas import tpu as pltpu
from jax.experimental.pallas import tpu_sc as plsc

NC = 2
NS = 16
NW = NC * NS
CHUNK = 128
ROW = 128


TR_LANES = 65536


def _tr_body(tt_ref, out_ref):
    x = tt_ref[...]
    for q in range(TR_LANES // 512):
        y = jnp.concatenate(
            [x[:, 512 * q + 128 * p:512 * q + 128 * (p + 1)]
             for p in range(4)], axis=0)
        out_ref[128 * q:128 * (q + 1), :] = jnp.transpose(y)


def _transpose_table(table_t):
    n_feat, vocab = table_t.shape
    grid = (pl.cdiv(vocab, TR_LANES),)
    out_rows = grid[0] * (TR_LANES // 4)
    return pl.pallas_call(
        _tr_body,
        grid=grid,
        in_specs=[pl.BlockSpec((n_feat, TR_LANES), lambda u: (0, u))],
        out_specs=pl.BlockSpec((TR_LANES // 4, ROW), lambda u: (u, 0)),
        out_shape=jax.ShapeDtypeStruct((out_rows, ROW), jnp.float32),
    )(table_t)


def _make_sc_gather(batch):
    b_per_w = batch // NW
    n_chunks = b_per_w // CHUNK
    mesh = plsc.VectorSubcoreMesh(core_axis_name="c", subcore_axis_name="s")

    @functools.partial(
        pl.kernel,
        mesh=mesh,
        out_type=jax.ShapeDtypeStruct((NW, b_per_w, ROW), jnp.float32),
        scratch_types=[
            pltpu.VMEM((n_chunks, CHUNK), jnp.int32),
            pltpu.VMEM((b_per_w, ROW), jnp.float32),
            pltpu.SemaphoreType.DMA,
        ],
    )
    def gather(table_hbm, idx_hbm, out_hbm, idx_v, rows_v, sem):
        wid = lax.axis_index("s") * NC + lax.axis_index("c")
        pltpu.sync_copy(idx_hbm.at[wid], idx_v)
        copies = [
            pltpu.async_copy(
                table_hbm.at[idx_v.at[j]],
                rows_v.at[pl.ds(j * CHUNK, CHUNK)],
                sem,
            )
            for j in range(n_chunks)
        ]
        for c in copies:
            c.wait()
        pltpu.sync_copy(rows_v, out_hbm.at[wid])

    return gather


def _mlp_body(xin_ref, x4_ref, w1x_ref, w1e_ref, b1_ref, w2_ref, b2_ref,
              out_ref):
    c0 = xin_ref[..., 0:1]
    s1 = jnp.floor(c0 * (1.0 / 128.0))
    sub = s1 - 4.0 * jnp.floor(s1 * 0.25)
    x4 = x4_ref[...]
    emb_dim = w1e_ref.shape[0]
    xemb = jnp.zeros((x4.shape[0], emb_dim), jnp.float32)
    for s in range(4):
        sel = sub == float(s)
        xemb = xemb + jnp.where(sel, x4[:, s * emb_dim:(s + 1) * emb_dim], 0.0)
    h = jnp.dot(xin_ref[...], w1x_ref[...], preferred_element_type=jnp.float32)
    h = h + jnp.dot(xemb, w1e_ref[...], preferred_element_type=jnp.float32)
    h = jnp.maximum(h + b1_ref[...], 0.0)
    o = jnp.dot(h, w2_ref[...], preferred_element_type=jnp.float32)
    out_ref[...] = jnp.maximum(o + b2_ref[...], 0.0)


def _mlp(inputs, x4, w1x, w1e, b1, w2, b2, block_m=1024):
    batch, n_feat = inputs.shape
    hidden = w1x.shape[1]
    out_dim = w2.shape[1]
    grid = (batch // block_m,)
    return pl.pallas_call(
        _mlp_body,
        grid=grid,
        in_specs=[
            pl.BlockSpec((block_m, n_feat), lambda i: (i, 0)),
            pl.BlockSpec((block_m, ROW), lambda i: (i, 0)),
            pl.BlockSpec((n_feat, hidden), lambda i: (0, 0)),
            pl.BlockSpec((w1e.shape[0], hidden), lambda i: (0, 0)),
            pl.BlockSpec((1, hidden), lambda i: (0, 0)),
            pl.BlockSpec((hidden, out_dim), lambda i: (0, 0)),
            pl.BlockSpec((1, out_dim), lambda i: (0, 0)),
        ],
        out_specs=pl.BlockSpec((block_m, out_dim), lambda i: (i, 0)),
        out_shape=jax.ShapeDtypeStruct((batch, out_dim), jnp.float32),
    )(inputs, x4, w1x, w1e, b1, w2, b2)


def kernel(inputs, emb_table, W1, b1, W2, b2):
    batch, n_feat = inputs.shape
    vocab, emb_dim = emb_table.shape
    hidden = W1.shape[1]

    idx = inputs[:, 0].astype(jnp.int32)
    hi = (((idx >> 9) << 7) | (idx & 127)).reshape(
        NW, batch // NW // CHUNK, CHUNK)
    table4 = _transpose_table(emb_table.T)
    x4 = _make_sc_gather(batch)(table4, hi)
    x4 = x4.reshape(batch, ROW)

    w1x = jnp.concatenate(
        [jnp.zeros((1, hidden), jnp.float32), W1[: n_feat - 1]], axis=0)
    w1e = W1[n_feat - 1:]
    return _mlp(inputs, x4, w1x, w1e, b1[None, :], W2, b2[None, :])
